# baseline (speedup 1.0000x reference)
import functools

import jax
import jax.numpy as jnp
from jax import lax
from jax.experimental import pallas as pl
from jax.experimental.pallas import tpu as pltpu
from jax.experimental.pallas import tpu_sc as plsc

_NC = 2
_NS = 16
_NW = _NC * _NS
_LANES = 16


def _tc_relayout(t_view, V, D):
    W = 2 * D
    BLK = 512
    grid = (V + BLK - 1) // BLK

    def body(in_ref, out_ref):
        x = in_ref[...]
        y = x.reshape(D, BLK // 2, 2).transpose(1, 0, 2)
        out_ref[...] = y.reshape(BLK // 2, W)

    return pl.pallas_call(
        body,
        grid=(grid,),
        in_specs=[pl.BlockSpec((D, BLK), lambda i: (0, i))],
        out_specs=pl.BlockSpec((BLK // 2, W), lambda i: (i, 0)),
        out_shape=jax.ShapeDtypeStruct(((V + 1) // 2, W), jnp.float32),
    )(t_view)


def _sc_relayout(t_view, tail, V, D):
    W = 2 * D
    nfull = V // 128
    tail_rows = (V - nfull * 128) // 2
    per_w = nfull // _NW + 1
    mesh = plsc.VectorSubcoreMesh(core_axis_name="c", subcore_axis_name="s")

    @functools.partial(
        pl.kernel,
        out_type=jax.ShapeDtypeStruct((V // 2, W), jnp.float32),
        mesh=mesh,
        compiler_params=pltpu.CompilerParams(needs_layout_passes=False),
        scratch_types=[
            pltpu.VMEM((2, D, 129), jnp.float32),
            pltpu.VMEM((2, 64, W), jnp.float32),
            pltpu.SemaphoreType.DMA,
            pltpu.SemaphoreType.DMA,
        ],
    )
    def relayout_kernel(t_hbm, tail_hbm, out_hbm, in_b, out_b, sem_i, sem_o):
        wid = lax.axis_index("s") * _NC + lax.axis_index("c")
        lane = lax.iota(jnp.int32, _LANES)
        dvecs = [(c0 + lane) >> 1 for c0 in range(0, W, _LANES)]
        svec = lane & 1

        def fire_in(c):
            cid = wid + c * _NW

            @pl.when(cid < nfull)
            def _():
                src = pl.multiple_of(cid * 128, 128)
                pltpu.async_copy(t_hbm.at[:, pl.ds(src, 128)],
                                 in_b.at[c % 2, :, pl.ds(0, 128)], sem_i)

        def block_body(c, _):
            cid = wid + c * _NW
            fire_in(c + 1)

            @pl.when(cid < nfull)
            def _():
                b = c % 2
                pltpu.make_async_copy(t_hbm.at[:, pl.ds(0, 128)],
                                      in_b.at[b, :, pl.ds(0, 128)],
                                      sem_i).wait()

                @pl.when(c >= 2)
                def _():
                    old = pl.multiple_of((cid - 2 * _NW) * 64, 64)
                    pltpu.make_async_copy(
                        out_b.at[b], out_hbm.at[pl.ds(old, 64), :],
                        sem_o).wait()

                inb, outb = in_b.at[b], out_b.at[b]

                @plsc.parallel_loop(0, 64, unroll=4)
                def row_body(p):
                    t = 2 * p + svec
                    for j, c0 in enumerate(range(0, W, _LANES)):
                        outb[p, pl.ds(c0, _LANES)] = plsc.load_gather(
                            inb, [dvecs[j], t])
                dst = pl.multiple_of(cid * 64, 64)
                pltpu.async_copy(outb, out_hbm.at[pl.ds(dst, 64), :], sem_o)

            return 0

        fire_in(0)
        lax.fori_loop(0, per_w, block_body, 0)

        def drain_body(c, _):
            cid = wid + c * _NW

            @pl.when((cid < nfull) & (cid + 2 * _NW >= nfull))
            def _():
                dst = pl.multiple_of(cid * 64, 64)
                pltpu.make_async_copy(
                    out_b.at[c % 2], out_hbm.at[pl.ds(dst, 64), :],
                    sem_o).wait()
            return 0

        lax.fori_loop(0, per_w, drain_body, 0)

        @pl.when(wid == 0)
        def _():
            pltpu.sync_copy(tail_hbm, out_b.at[0, pl.ds(0, tail_rows), :])
            pltpu.sync_copy(out_b.at[0, pl.ds(0, tail_rows), :],
                            out_hbm.at[pl.ds(nfull * 64, tail_rows), :])

    return relayout_kernel(t_view, tail)


def _sc_scores(pos_u, pos_v, neg_t, u_pair, v_pair, B, K, D, chunk):
    per_w = B // _NW
    nchunks = per_w // chunk
    ngroups = chunk // _LANES
    W = 2 * D
    mesh = plsc.VectorSubcoreMesh(core_axis_name="c", subcore_axis_name="s")

    @functools.partial(
        pl.kernel,
        out_type=jax.ShapeDtypeStruct(((1 + K) * B,), jnp.float32),
        mesh=mesh,
        compiler_params=pltpu.CompilerParams(needs_layout_passes=False),
        scratch_types=[
            pltpu.VMEM((chunk,), jnp.int32),
            pltpu.VMEM((chunk,), jnp.int32),
            pltpu.VMEM((K, chunk), jnp.int32),
            pltpu.VMEM((chunk,), jnp.int32),
            pltpu.VMEM((chunk,), jnp.int32),
            pltpu.VMEM((K, chunk), jnp.int32),
            pltpu.VMEM((chunk, 2 * D), jnp.float32),
            pltpu.VMEM((chunk, 2 * D), jnp.float32),
            pltpu.VMEM((K, chunk, 2 * D), jnp.float32),
            pltpu.VMEM((1 + K, chunk), jnp.float32),
            pltpu.SemaphoreType.DMA,
        ],
    )
    def scores_kernel(pos_u_hbm, pos_v_hbm, neg_t_hbm, u_hbm, v_hbm, out_hbm,
                      idx_u, idx_v, idx_n, idx_uh, idx_vh, idx_nh,
                      u_rows, v_rows, n_rows, scores, sem):
        wid = lax.axis_index("s") * _NC + lax.axis_index("c")
        base = wid * per_w
        zero16 = jnp.zeros((_LANES,), jnp.float32)

        def halve(src, dst):
            for g in range(ngroups):
                sl = pl.ds(g * _LANES, _LANES)
                dst[sl] = lax.shift_right_logical(src[sl], 1)

        def chunk_body(c, _):
            off = pl.multiple_of(base + c * chunk, chunk)
            pltpu.sync_copy(pos_u_hbm.at[pl.ds(off, chunk)], idx_u)
            pltpu.sync_copy(pos_v_hbm.at[pl.ds(off, chunk)], idx_v)
            pltpu.sync_copy(neg_t_hbm.at[:, pl.ds(off, chunk)], idx_n)
            halve(idx_u, idx_uh)
            halve(idx_v, idx_vh)
            for k in range(K):
                halve(idx_n.at[k], idx_nh.at[k])
            cps = [pltpu.async_copy(u_hbm.at[idx_uh], u_rows, sem),
                   pltpu.async_copy(v_hbm.at[idx_vh], v_rows, sem)]
            for k in range(K):
                cps.append(
                    pltpu.async_copy(v_hbm.at[idx_nh.at[k]], n_rows.at[k], sem))
            for cp in cps:
                cp.wait()

            def group_body(g, _):
                sl = pl.ds(g * _LANES, _LANES)
                row = g * _LANES + lax.iota(jnp.int32, _LANES)
                off_u = idx_u[sl] & 1
                off_v = idx_v[sl] & 1
                off_n = [idx_n[k, sl] & 1 for k in range(K)]
                accs = [zero16] * (1 + K)
                for d in range(D):
                    u_col = plsc.load_gather(u_rows, [row, off_u + 2 * d])
                    accs[0] = accs[0] + u_col * plsc.load_gather(
                        v_rows, [row, off_v + 2 * d])
                    for k in range(K):
                        accs[1 + k] = accs[1 + k] + u_col * plsc.load_gather(
                            n_rows.at[k], [row, off_n[k] + 2 * d])
                for r in range(1 + K):
                    scores[r, sl] = accs[r]
                return 0

            lax.fori_loop(0, ngroups, group_body, 0)
            for r in range(1 + K):
                pltpu.sync_copy(scores.at[r], out_hbm.at[pl.ds(r * B + off, chunk)])
            return 0

        lax.fori_loop(0, nchunks, chunk_body, 0)

    return scores_kernel(pos_u, pos_v, neg_t, u_pair, v_pair)


def _loss_body(s_ref, o_ref):
    s = s_ref[...]
    pos = s[0:1, :]
    neg = s[1:, :]

    def logsig(x):
        return jnp.minimum(x, 0.0) - jnp.log1p(jnp.exp(-jnp.abs(x)))

    total = jnp.sum(logsig(pos)) + jnp.sum(logsig(-neg))
    o_ref[...] = (-total).reshape(1, 1)


def kernel(pos_u, pos_v, neg_v, u_table, v_table):
    B = pos_u.shape[0]
    K = neg_v.shape[1]
    V, D = u_table.shape
    pos_u = pos_u.astype(jnp.int32)
    pos_v = pos_v.astype(jnp.int32)
    neg_t = neg_v.astype(jnp.int32).T

    nfull = (V // 128) * 128
    v_tail = (v_table[nfull:].reshape(-1, 2, D)
              .transpose(0, 2, 1).reshape(-1, 2 * D))
    u_pair = _tc_relayout(u_table.T, V, D)
    v_pair = _sc_relayout(v_table.T, v_tail, V, D)

    scores = _sc_scores(pos_u, pos_v, neg_t, u_pair, v_pair, B, K, D,
                        chunk=128).reshape(1 + K, B)

    loss = pl.pallas_call(
        _loss_body,
        out_shape=jax.ShapeDtypeStruct((1, 1), jnp.float32),
    )(scores)
    return loss[0, 0]


# speedup vs baseline: 5.2709x; 5.2709x over previous
import functools

import jax
import jax.numpy as jnp
from jax import lax
from jax.experimental import pallas as pl
from jax.experimental.pallas import tpu as pltpu
from jax.experimental.pallas import tpu_sc as plsc

_NC = 2
_NS = 16
_NW = _NC * _NS
_LANES = 16


def _sc_relayout(u_t, v_t, u_tail, v_tail, V, D):
    W = 2 * D
    nfull = V // 128
    tail_rows = (V - nfull * 128) // 2
    per_w = nfull // _NW + 1
    mesh = plsc.VectorSubcoreMesh(core_axis_name="c", subcore_axis_name="s")

    @functools.partial(
        pl.kernel,
        out_type=(jax.ShapeDtypeStruct((V // 2, W), jnp.float32),
                  jax.ShapeDtypeStruct((V // 2, W), jnp.float32)),
        mesh=mesh,
        compiler_params=pltpu.CompilerParams(needs_layout_passes=False),
        scratch_types=[
            pltpu.VMEM((2, D, 128), jnp.float32),
            pltpu.VMEM((2, D, 128), jnp.float32),
            pltpu.VMEM((2, 64, 131), jnp.float32),
            pltpu.VMEM((2, 64, 131), jnp.float32),
            pltpu.SemaphoreType.DMA,
            pltpu.SemaphoreType.DMA,
        ],
    )
    def relayout_kernel(u_t_hbm, v_t_hbm, u_tail_hbm, v_tail_hbm,
                        u_out_hbm, v_out_hbm, in_u, in_v, out_u, out_v,
                        sem_i, sem_o):
        wid = lax.axis_index("s") * _NC + lax.axis_index("c")
        lane = lax.iota(jnp.int32, _LANES)
        rvecs = [(t0 + lane) >> 1 for t0 in range(0, 128, _LANES)]
        pbase = (lane & 1) * D

        def fire_in(c):
            cid = wid + c * _NW

            @pl.when(cid < nfull)
            def _():
                src = pl.multiple_of(cid * 128, 128)
                b = c % 2
                pltpu.async_copy(u_t_hbm.at[:, pl.ds(src, 128)],
                                 in_u.at[b], sem_i)
                pltpu.async_copy(v_t_hbm.at[:, pl.ds(src, 128)],
                                 in_v.at[b], sem_i)

        def block_body(c, _):
            cid = wid + c * _NW
            fire_in(c + 1)

            @pl.when(cid < nfull)
            def _():
                b = c % 2
                pltpu.make_async_copy(u_t_hbm.at[:, pl.ds(0, 128)],
                                      in_u.at[b], sem_i).wait()
                pltpu.make_async_copy(v_t_hbm.at[:, pl.ds(0, 128)],
                                      in_v.at[b], sem_i).wait()
                @pl.when(c >= 2)
                def _():
                    old = pl.multiple_of((cid - 2 * _NW) * 64, 64)
                    pltpu.make_async_copy(
                        out_u.at[b, :, pl.ds(0, W)],
                        u_out_hbm.at[pl.ds(old, 64), :], sem_o).wait()
                    pltpu.make_async_copy(
                        out_v.at[b, :, pl.ds(0, W)],
                        v_out_hbm.at[pl.ds(old, 64), :], sem_o).wait()

                inub, invb = in_u.at[b], in_v.at[b]
                outub, outvb = out_u.at[b], out_v.at[b]

                @plsc.parallel_loop(0, D, unroll=4)
                def row_body(d):
                    cvec = pbase + d
                    for inb, outb in ((inub, outub), (invb, outvb)):
                        for j, t0 in enumerate(range(0, 128, _LANES)):
                            plsc.store_scatter(outb, [rvecs[j], cvec],
                                               inb[d, pl.ds(t0, _LANES)])
                dst = pl.multiple_of(cid * 64, 64)
                pltpu.async_copy(out_u.at[b, :, pl.ds(0, W)],
                                 u_out_hbm.at[pl.ds(dst, 64), :], sem_o)
                pltpu.async_copy(out_v.at[b, :, pl.ds(0, W)],
                                 v_out_hbm.at[pl.ds(dst, 64), :], sem_o)

            return 0

        fire_in(0)
        lax.fori_loop(0, per_w, block_body, 0)

        def drain_body(c, _):
            cid = wid + c * _NW

            @pl.when((cid < nfull) & (cid + 2 * _NW >= nfull))
            def _():
                b = c % 2
                dst = pl.multiple_of(cid * 64, 64)
                pltpu.make_async_copy(
                    out_u.at[b, :, pl.ds(0, W)],
                    u_out_hbm.at[pl.ds(dst, 64), :], sem_o).wait()
                pltpu.make_async_copy(
                    out_v.at[b, :, pl.ds(0, W)],
                    v_out_hbm.at[pl.ds(dst, 64), :], sem_o).wait()
            return 0

        lax.fori_loop(0, per_w, drain_body, 0)

        @pl.when(wid == 0)
        def _():
            ut = out_u.at[0, pl.ds(0, tail_rows), pl.ds(0, W)]
            vt = out_v.at[0, pl.ds(0, tail_rows), pl.ds(0, W)]
            pltpu.sync_copy(u_tail_hbm, ut)
            pltpu.sync_copy(ut, u_out_hbm.at[pl.ds(nfull * 64, tail_rows), :])
            pltpu.sync_copy(v_tail_hbm, vt)
            pltpu.sync_copy(vt, v_out_hbm.at[pl.ds(nfull * 64, tail_rows), :])

    return relayout_kernel(u_t, v_t, u_tail, v_tail)


def _sc_scores(pos_u, pos_v, neg_t, u_pair, v_pair, B, K, D, chunk):
    per_w = B // _NW
    nchunks = per_w // chunk
    ngroups = chunk // _LANES
    W = 2 * D
    mesh = plsc.VectorSubcoreMesh(core_axis_name="c", subcore_axis_name="s")

    @functools.partial(
        pl.kernel,
        out_type=jax.ShapeDtypeStruct(((1 + K) * B,), jnp.float32),
        mesh=mesh,
        compiler_params=pltpu.CompilerParams(needs_layout_passes=False),
        scratch_types=[
            pltpu.VMEM((chunk,), jnp.int32),
            pltpu.VMEM((chunk,), jnp.int32),
            pltpu.VMEM((K, chunk), jnp.int32),
            pltpu.VMEM((chunk,), jnp.int32),
            pltpu.VMEM((chunk,), jnp.int32),
            pltpu.VMEM((K, chunk), jnp.int32),
            pltpu.VMEM((chunk, 2 * D), jnp.float32),
            pltpu.VMEM((chunk, 2 * D), jnp.float32),
            pltpu.VMEM((K, chunk, 2 * D), jnp.float32),
            pltpu.VMEM((1 + K, chunk), jnp.float32),
            pltpu.SemaphoreType.DMA,
        ],
    )
    def scores_kernel(pos_u_hbm, pos_v_hbm, neg_t_hbm, u_hbm, v_hbm, out_hbm,
                      idx_u, idx_v, idx_n, idx_uh, idx_vh, idx_nh,
                      u_rows, v_rows, n_rows, scores, sem):
        wid = lax.axis_index("s") * _NC + lax.axis_index("c")
        base = wid * per_w
        zero16 = jnp.zeros((_LANES,), jnp.float32)

        def halve(src, dst):
            for g in range(ngroups):
                sl = pl.ds(g * _LANES, _LANES)
                dst[sl] = lax.shift_right_logical(src[sl], 1)

        def chunk_body(c, _):
            off = pl.multiple_of(base + c * chunk, chunk)
            pltpu.sync_copy(pos_u_hbm.at[pl.ds(off, chunk)], idx_u)
            pltpu.sync_copy(pos_v_hbm.at[pl.ds(off, chunk)], idx_v)
            pltpu.sync_copy(neg_t_hbm.at[:, pl.ds(off, chunk)], idx_n)
            halve(idx_u, idx_uh)
            halve(idx_v, idx_vh)
            for k in range(K):
                halve(idx_n.at[k], idx_nh.at[k])
            cps = [pltpu.async_copy(u_hbm.at[idx_uh], u_rows, sem),
                   pltpu.async_copy(v_hbm.at[idx_vh], v_rows, sem)]
            for k in range(K):
                cps.append(
                    pltpu.async_copy(v_hbm.at[idx_nh.at[k]], n_rows.at[k], sem))
            for cp in cps:
                cp.wait()

            def group_body(g, _):
                sl = pl.ds(g * _LANES, _LANES)
                row = g * _LANES + lax.iota(jnp.int32, _LANES)
                off_u = (idx_u[sl] & 1) * D
                off_v = (idx_v[sl] & 1) * D
                off_n = [(idx_n[k, sl] & 1) * D for k in range(K)]
                accs = [zero16] * (1 + K)
                for d in range(D):
                    u_col = plsc.load_gather(u_rows, [row, off_u + d])
                    accs[0] = accs[0] + u_col * plsc.load_gather(
                        v_rows, [row, off_v + d])
                    for k in range(K):
                        accs[1 + k] = accs[1 + k] + u_col * plsc.load_gather(
                            n_rows.at[k], [row, off_n[k] + d])
                for r in range(1 + K):
                    scores[r, sl] = accs[r]
                return 0

            lax.fori_loop(0, ngroups, group_body, 0)
            for r in range(1 + K):
                pltpu.sync_copy(scores.at[r], out_hbm.at[pl.ds(r * B + off, chunk)])
            return 0

        lax.fori_loop(0, nchunks, chunk_body, 0)

    return scores_kernel(pos_u, pos_v, neg_t, u_pair, v_pair)


def _loss_body(s_ref, o_ref):
    s = s_ref[...]
    pos = s[0:1, :]
    neg = s[1:, :]

    def logsig(x):
        return jnp.minimum(x, 0.0) - jnp.log1p(jnp.exp(-jnp.abs(x)))

    total = jnp.sum(logsig(pos)) + jnp.sum(logsig(-neg))
    o_ref[...] = (-total).reshape(1, 1)


def kernel(pos_u, pos_v, neg_v, u_table, v_table):
    B = pos_u.shape[0]
    K = neg_v.shape[1]
    V, D = u_table.shape
    pos_u = pos_u.astype(jnp.int32)
    pos_v = pos_v.astype(jnp.int32)
    neg_t = neg_v.astype(jnp.int32).T

    nfull = (V // 128) * 128
    u_tail = u_table[nfull:].reshape(-1, 2 * D)
    v_tail = v_table[nfull:].reshape(-1, 2 * D)
    u_pair, v_pair = _sc_relayout(u_table.T, v_table.T, u_tail, v_tail, V, D)

    scores = _sc_scores(pos_u, pos_v, neg_t, u_pair, v_pair, B, K, D,
                        chunk=128).reshape(1 + K, B)

    loss = pl.pallas_call(
        _loss_body,
        out_shape=jax.ShapeDtypeStruct((1, 1), jnp.float32),
    )(scores)
    return loss[0, 0]


# speedup vs baseline: 10.1451x; 1.9247x over previous
import functools

import jax
import jax.numpy as jnp
from jax import lax
from jax.experimental import pallas as pl
from jax.experimental.pallas import tpu as pltpu
from jax.experimental.pallas import tpu_sc as plsc

_NC = 2
_NS = 16
_NW = _NC * _NS
_LANES = 16


def _sc_scores(pos_u, pos_v, neg_t, u_rm, v_rm, B, K, D, chunk):
    per_w = B // _NW
    nchunks = per_w // chunk
    ngroups = chunk // _LANES
    mesh = plsc.VectorSubcoreMesh(core_axis_name="c", subcore_axis_name="s")

    @functools.partial(
        pl.kernel,
        out_type=jax.ShapeDtypeStruct(((1 + K) * B,), jnp.float32),
        mesh=mesh,
        compiler_params=pltpu.CompilerParams(needs_layout_passes=False),
        scratch_types=[
            pltpu.VMEM((per_w,), jnp.int32),
            pltpu.VMEM((per_w,), jnp.int32),
            pltpu.VMEM((K, per_w), jnp.int32),
            pltpu.VMEM((chunk, 8, D), jnp.float32),
            pltpu.VMEM((chunk, 8, D), jnp.float32),
            pltpu.VMEM((K * chunk, 8, D), jnp.float32),
            pltpu.VMEM((1 + K, per_w), jnp.float32),
            pltpu.SemaphoreType.DMA,
        ],
    )
    def scores_kernel(pos_u_hbm, pos_v_hbm, neg_t_hbm, u_hbm, v_hbm, out_hbm,
                      aidx_u, aidx_v, aidx_n, u_blk, v_blk, n_blk, scores,
                      sem):
        wid = lax.axis_index("s") * _NC + lax.axis_index("c")
        base = wid * per_w
        lane = lax.iota(jnp.int32, _LANES)
        zero16 = jnp.zeros((_LANES,), jnp.float32)

        pltpu.sync_copy(pos_u_hbm.at[pl.ds(pl.multiple_of(base, per_w),
                                           per_w)], aidx_u)
        pltpu.sync_copy(pos_v_hbm.at[pl.ds(pl.multiple_of(base, per_w),
                                           per_w)], aidx_v)
        pltpu.sync_copy(neg_t_hbm.at[:, pl.ds(pl.multiple_of(base, per_w),
                                              per_w)], aidx_n)

        def row_group(tbl, i):
            i8 = pl.multiple_of((i >> 3) * 8, 8)
            return tbl.at[pl.ds(i8, 8), :]

        def chunk_body(c, _):
            coff = c * chunk
            for g in range(ngroups):
                sl = pl.ds(coff + g * _LANES, _LANES)
                iu_vec = aidx_u[sl]
                iv_vec = aidx_v[sl]
                in_vecs = [aidx_n[k, sl] for k in range(K)]
                for i in range(_LANES):
                    e = g * _LANES + i
                    pltpu.async_copy(row_group(u_hbm, iu_vec[i]),
                                     u_blk.at[e], sem)
                    pltpu.async_copy(row_group(v_hbm, iv_vec[i]),
                                     v_blk.at[e], sem)
                    for k in range(K):
                        pltpu.async_copy(row_group(v_hbm, in_vecs[k][i]),
                                         n_blk.at[k * chunk + e], sem)
            for e in range(chunk):
                pltpu.make_async_copy(u_hbm.at[pl.ds(0, 8), :],
                                      u_blk.at[e], sem).wait()
                pltpu.make_async_copy(u_hbm.at[pl.ds(0, 8), :],
                                      v_blk.at[e], sem).wait()
                for k in range(K):
                    pltpu.make_async_copy(u_hbm.at[pl.ds(0, 8), :],
                                          n_blk.at[k * chunk + e], sem).wait()

            for g in range(ngroups):
                sl = pl.ds(coff + g * _LANES, _LANES)
                evec = g * _LANES + lane
                ru = aidx_u[sl] & 7
                rv = aidx_v[sl] & 7
                rn = [aidx_n[k, sl] & 7 for k in range(K)]
                accs = [zero16] * (1 + K)
                for d in range(D):
                    dv = jnp.full((_LANES,), d, jnp.int32)
                    u_col = plsc.load_gather(u_blk, [evec, ru, dv])
                    accs[0] = accs[0] + u_col * plsc.load_gather(
                        v_blk, [evec, rv, dv])
                    for k in range(K):
                        accs[1 + k] = accs[1 + k] + u_col * plsc.load_gather(
                            n_blk, [k * chunk + evec, rn[k], dv])
                for r in range(1 + K):
                    scores[r, sl] = accs[r]
            return 0

        lax.fori_loop(0, nchunks, chunk_body, 0)
        for r in range(1 + K):
            pltpu.sync_copy(scores.at[r],
                            out_hbm.at[pl.ds(r * B + base, per_w)])

    return scores_kernel(pos_u, pos_v, neg_t, u_rm, v_rm)


def _loss_body(s_ref, o_ref):
    s = s_ref[...]
    pos = s[0:1, :]
    neg = s[1:, :]

    def logsig(x):
        return jnp.minimum(x, 0.0) - jnp.log1p(jnp.exp(-jnp.abs(x)))

    total = jnp.sum(logsig(pos)) + jnp.sum(logsig(-neg))
    o_ref[...] = (-total).reshape(1, 1)


def kernel(pos_u, pos_v, neg_v, u_table, v_table):
    B = pos_u.shape[0]
    K = neg_v.shape[1]
    V, D = u_table.shape
    pos_u = pos_u.astype(jnp.int32)
    pos_v = pos_v.astype(jnp.int32)
    neg_t = neg_v.astype(jnp.int32).T

    scores = _sc_scores(pos_u, pos_v, neg_t, u_table, v_table, B, K, D,
                        chunk=16).reshape(1 + K, B)

    loss = pl.pallas_call(
        _loss_body,
        out_shape=jax.ShapeDtypeStruct((1, 1), jnp.float32),
    )(scores)
    return loss[0, 0]
